# baseline (speedup 1.0000x reference)
import jax
import jax.numpy as jnp
from jax.experimental import pallas as pl
from jax.experimental.pallas import tpu as pltpu

DIM = 512
CHUNK = 8192
NBUF = 3
BPC = 4


def _pool_kernel(b_ref, x_hbm, w_ref, out_ref, xbuf, sems):
    p = pl.program_id(0)
    n = x_hbm.shape[1]
    ncs = n // CHUNK
    total = BPC * ncs

    def copy_for(c):
        gb = p * BPC + c // ncs
        off = (c % ncs) * CHUNK
        slot = c % NBUF
        return pltpu.make_async_copy(
            x_hbm.at[gb, pl.ds(off, CHUNK), :],
            xbuf.at[slot],
            sems.at[slot],
        )

    for c in range(NBUF - 1):
        copy_for(c).start()

    w = w_ref[...]
    ba = b_ref[0, 0]
    bg = b_ref[0, 1]
    for b in range(BPC):
        acc = jnp.zeros((1, DIM), jnp.float32)
        ssum = jnp.float32(0.0)
        for k in range(ncs):
            c = b * ncs + k
            if c + NBUF - 1 < total:
                copy_for(c + NBUF - 1).start()
            copy_for(c).wait()
            x = xbuf[c % NBUF]
            proj = jnp.dot(x, w, preferred_element_type=jnp.float32)
            a = jnp.tanh(proj[:, 0:1] + ba)
            g = jax.nn.sigmoid(proj[:, 1:2] + bg)
            e = jnp.exp(a * g)
            acc = acc + jnp.sum(e * x, axis=0, keepdims=True)
            ssum = ssum + jnp.sum(e)
        out_ref[b, 0, :] = (acc * (1.0 / ssum))[0]


def kernel(x, W_a, b_a, W_g, b_g):
    B, N, D = x.shape
    w = jnp.concatenate([W_a, W_g], axis=1)
    biases = jnp.stack([b_a[0], b_g[0]]).reshape(1, 2)

    out = pl.pallas_call(
        _pool_kernel,
        grid=(B // BPC,),
        in_specs=[
            pl.BlockSpec(memory_space=pltpu.SMEM),
            pl.BlockSpec(memory_space=pl.ANY),
            pl.BlockSpec((D, 2), lambda p: (0, 0)),
        ],
        out_specs=pl.BlockSpec((BPC, 1, D), lambda p: (p, 0, 0)),
        out_shape=jax.ShapeDtypeStruct((B, 1, D), jnp.float32),
        scratch_shapes=[
            pltpu.VMEM((NBUF, CHUNK, DIM), jnp.float32),
            pltpu.SemaphoreType.DMA((NBUF,)),
        ],
        compiler_params=pltpu.CompilerParams(
            dimension_semantics=("parallel",),
        ),
    )(biases, x, w)
    return out


# speedup vs baseline: 1.6272x; 1.6272x over previous
import jax
import jax.numpy as jnp
from jax.experimental import pallas as pl
from jax.experimental.pallas import tpu as pltpu

DIM = 512
BN = 8192


def _pool_kernel(b_ref, x_ref, w_ref, out_ref, acc_ref, ssum_ref):
    j = pl.program_id(1)
    nj = pl.num_programs(1)

    @pl.when(j == 0)
    def _init():
        acc_ref[...] = jnp.zeros_like(acc_ref)
        ssum_ref[0, 0] = 0.0

    x = x_ref[0]
    proj = jnp.dot(x, w_ref[...], preferred_element_type=jnp.float32)
    a = jnp.tanh(proj[:, 0:1] + b_ref[0, 0])
    g = jax.nn.sigmoid(proj[:, 1:2] + b_ref[0, 1])
    e = jnp.exp(a * g)

    acc_ref[...] += jnp.sum(e * x, axis=0, keepdims=True)
    ssum_ref[0, 0] += jnp.sum(e)

    @pl.when(j == nj - 1)
    def _finish():
        out_ref[0] = acc_ref[...] / ssum_ref[0, 0]


def kernel(x, W_a, b_a, W_g, b_g):
    B, N, D = x.shape
    w = jnp.concatenate([W_a, W_g], axis=1)
    biases = jnp.stack([b_a[0], b_g[0]]).reshape(1, 2)

    nj = N // BN
    out = pl.pallas_call(
        _pool_kernel,
        grid=(B, nj),
        in_specs=[
            pl.BlockSpec(memory_space=pltpu.SMEM),
            pl.BlockSpec((1, BN, D), lambda b, j: (b, j, 0)),
            pl.BlockSpec((D, 2), lambda b, j: (0, 0)),
        ],
        out_specs=pl.BlockSpec((1, 1, D), lambda b, j: (b, 0, 0)),
        out_shape=jax.ShapeDtypeStruct((B, 1, D), jnp.float32),
        scratch_shapes=[
            pltpu.VMEM((1, D), jnp.float32),
            pltpu.SMEM((1, 1), jnp.float32),
        ],
        compiler_params=pltpu.CompilerParams(
            dimension_semantics=("parallel", "arbitrary"),
        ),
    )(biases, x, w)
    return out
